# Initial kernel scaffold; baseline (speedup 1.0000x reference)
import jax
import jax.numpy as jnp
from jax import lax
from jax.experimental import pallas as pl
from jax.experimental.pallas import tpu as pltpu
from jax.experimental.pallas import tpu_sc as plsc

N = 10000
E = 320000
D = 128
H = 128
C = 16
G = 64

NC = 2
NS = 16
NW = NC * NS
CHUNK = 128
K_CH = 79
E_PAD = NW * CHUNK * K_CH
DUMMY = N

BLK = 256
N_PAD = 10240
NBLK = N_PAD // BLK
ROWS_PER_TILE = N_PAD // NS


def _make_sc_agg(F):
    mesh = plsc.VectorSubcoreMesh(
        core_axis_name="c", subcore_axis_name="s",
        num_cores=NC, num_subcores=NS)

    def body(tbl_hbm, src_hbm, dst_hbm, zer_hbm, out_hbm,
             src_v, dst_v, rows_v, agg_sh):
        c = lax.axis_index("c")
        s = lax.axis_index("s")
        wid = s * NC + c
        r0 = s * ROWS_PER_TILE
        pltpu.sync_copy(zer_hbm, agg_sh.at[pl.ds(r0, ROWS_PER_TILE)])
        pltpu.sync_copy(src_hbm.at[wid], src_v)
        pltpu.sync_copy(dst_hbm.at[wid], dst_v)
        plsc.subcore_barrier()

        def chunk(j, carry):
            pltpu.sync_copy(tbl_hbm.at[src_v.at[j]], rows_v)
            pltpu.sync_copy(rows_v, agg_sh.at[dst_v.at[j]], add=True)
            return carry

        lax.fori_loop(0, K_CH, chunk, 0)
        plsc.subcore_barrier()
        pltpu.sync_copy(agg_sh.at[pl.ds(r0, ROWS_PER_TILE)],
                        out_hbm.at[c, pl.ds(r0, ROWS_PER_TILE)])

    return pl.kernel(
        body,
        out_type=jax.ShapeDtypeStruct((NC, N_PAD, F), jnp.float32),
        mesh=mesh,
        scratch_types=[
            pltpu.VMEM((K_CH, CHUNK), jnp.int32),
            pltpu.VMEM((K_CH, CHUNK), jnp.int32),
            pltpu.VMEM((CHUNK, F), jnp.float32),
            pltpu.VMEM_SHARED((N_PAD, F), jnp.float32),
        ],
    )


_agg128 = _make_sc_agg(128)
_agg16 = _make_sc_agg(16)


def _prep_body(x_ref, w_ref, dp_ref, hs_ref, dv_ref):
    deg = dp_ref[0, :, 0:1] + dp_ref[1, :, 0:1] + 1.0
    dinv = lax.rsqrt(deg)
    h = jnp.dot(x_ref[...], w_ref[...], preferred_element_type=jnp.float32)
    hs_ref[...] = h * dinv
    dv_ref[...] = jnp.broadcast_to(dinv, (BLK, 128))


def _prep(x_pad, W_in, degp):
    return pl.pallas_call(
        _prep_body,
        grid=(NBLK,),
        in_specs=[
            pl.BlockSpec((BLK, D), lambda i: (i, 0)),
            pl.BlockSpec((D, H), lambda i: (0, 0)),
            pl.BlockSpec((NC, BLK, 16), lambda i: (0, i, 0)),
        ],
        out_specs=[
            pl.BlockSpec((BLK, H), lambda i: (i, 0)),
            pl.BlockSpec((BLK, 128), lambda i: (i, 0)),
        ],
        out_shape=[
            jax.ShapeDtypeStruct((N_PAD, H), jnp.float32),
            jax.ShapeDtypeStruct((N_PAD, 128), jnp.float32),
        ],
    )(x_pad, W_in, degp)


def _mid_body(p_ref, hs_ref, dv_ref, b_ref, w_ref, o_ref):
    dinv = dv_ref[:, 0:1]
    a = (p_ref[0] + p_ref[1] + hs_ref[...]) * dinv + b_ref[...]
    a = jnp.maximum(a, 0.0)
    o_ref[...] = jnp.dot(a, w_ref[...],
                         preferred_element_type=jnp.float32) * dinv


def _mid(p, hs_prev, dinvb, b2d, W, F_out):
    return pl.pallas_call(
        _mid_body,
        grid=(NBLK,),
        in_specs=[
            pl.BlockSpec((NC, BLK, H), lambda i: (0, i, 0)),
            pl.BlockSpec((BLK, H), lambda i: (i, 0)),
            pl.BlockSpec((BLK, 128), lambda i: (i, 0)),
            pl.BlockSpec((1, H), lambda i: (0, 0)),
            pl.BlockSpec((H, F_out), lambda i: (0, 0)),
        ],
        out_specs=pl.BlockSpec((BLK, F_out), lambda i: (i, 0)),
        out_shape=jax.ShapeDtypeStruct((N_PAD, F_out), jnp.float32),
    )(p, hs_prev, dinvb, b2d, W)


def _pool_body(p_ref, hs_ref, dv_ref, b_ref, bt_ref, o_ref, sums_ref, cnt_ref):
    i = pl.program_id(0)

    @pl.when(i == 0)
    def _():
        sums_ref[...] = jnp.zeros_like(sums_ref)
        cnt_ref[...] = jnp.zeros_like(cnt_ref)

    dinv = dv_ref[:, 0:1]
    h3 = (p_ref[0] + p_ref[1] + hs_ref[...]) * dinv + b_ref[...]
    seg = bt_ref[0, 0, :]
    oh = (seg[:, None] == lax.broadcasted_iota(jnp.int32, (1, G), 1)
          ).astype(jnp.float32)
    sums_ref[...] += lax.dot_general(
        oh, h3, (((0,), (0,)), ((), ())), preferred_element_type=jnp.float32)
    cnt_ref[...] += lax.dot_general(
        oh, jnp.ones_like(h3), (((0,), (0,)), ((), ())),
        preferred_element_type=jnp.float32)

    @pl.when(i == pl.num_programs(0) - 1)
    def _():
        pooled = sums_ref[...] / jnp.maximum(cnt_ref[...], 1.0)
        m = jnp.max(pooled, axis=1, keepdims=True)
        ex = jnp.exp(pooled - m)
        o_ref[...] = (pooled - m) - jnp.log(jnp.sum(ex, axis=1, keepdims=True))


def _pool(p3, hs3, dinvb, b2d, batch3d):
    return pl.pallas_call(
        _pool_body,
        grid=(NBLK,),
        in_specs=[
            pl.BlockSpec((NC, BLK, C), lambda i: (0, i, 0)),
            pl.BlockSpec((BLK, C), lambda i: (i, 0)),
            pl.BlockSpec((BLK, 128), lambda i: (i, 0)),
            pl.BlockSpec((1, C), lambda i: (0, 0)),
            pl.BlockSpec((1, 1, BLK), lambda i: (i, 0, 0)),
        ],
        out_specs=pl.BlockSpec((G, C), lambda i: (0, 0)),
        out_shape=jax.ShapeDtypeStruct((G, C), jnp.float32),
        scratch_shapes=[
            pltpu.VMEM((G, C), jnp.float32),
            pltpu.VMEM((G, C), jnp.float32),
        ],
    )(p3, hs3, dinvb, b2d, batch3d)


def kernel(x, edge_index, batch, W_in, b_in, W_h, b_h, W_out, b_out):
    x_pad = jnp.zeros((N_PAD, D), jnp.float32).at[:N].set(x)
    pad_e = jnp.full((E_PAD - E,), DUMMY, jnp.int32)
    src = jnp.concatenate([edge_index[0], pad_e]).reshape(NW, K_CH, CHUNK)
    dst = jnp.concatenate([edge_index[1], pad_e]).reshape(NW, K_CH, CHUNK)
    batch3d = jnp.concatenate(
        [batch, jnp.full((N_PAD - N,), G, jnp.int32)]).reshape(NBLK, 1, BLK)
    zer128 = jnp.zeros((ROWS_PER_TILE, 128), jnp.float32)
    zer16 = jnp.zeros((ROWS_PER_TILE, 16), jnp.float32)
    ones_tbl = jnp.ones((N_PAD, 16), jnp.float32)

    degp = _agg16(ones_tbl, src, dst, zer16)
    hs1, dinvb = _prep(x_pad, W_in, degp)
    p1 = _agg128(hs1, src, dst, zer128)
    hs2 = _mid(p1, hs1, dinvb, b_in.reshape(1, H), W_h, H)
    p2 = _agg128(hs2, src, dst, zer128)
    hs3 = _mid(p2, hs2, dinvb, b_h.reshape(1, H), W_out, C)
    p3 = _agg16(hs3, src, dst, zer16)
    return _pool(p3, hs3, dinvb, b_out.reshape(1, C), batch3d)


# speedup vs baseline: 11.5349x; 11.5349x over previous
import jax
import jax.numpy as jnp
from jax import lax
from jax.experimental import pallas as pl
from jax.experimental.pallas import tpu as pltpu
from jax.experimental.pallas import tpu_sc as plsc

N = 10000
E = 320000
D = 128
H = 128
C = 16
G = 64

NC = 2
NS = 16
NW = NC * NS
CHUNK = 128
K_CH = 79
E_PAD = NW * CHUNK * K_CH
DUMMY = N

BLK = 256
N_PAD = 10240
NBLK = N_PAD // BLK
ROWS_PER_TILE = N_PAD // NS


def _make_sc_agg(F):
    mesh = plsc.VectorSubcoreMesh(
        core_axis_name="c", subcore_axis_name="s",
        num_cores=NC, num_subcores=NS)

    def body(tbl_hbm, src_hbm, dst_hbm, zer_hbm, out_hbm,
             src_v, dst_v, rows_v, agg_sh):
        c = lax.axis_index("c")
        s = lax.axis_index("s")
        wid = s * NC + c
        r0 = s * ROWS_PER_TILE
        pltpu.sync_copy(zer_hbm, agg_sh.at[pl.ds(r0, ROWS_PER_TILE)])
        pltpu.sync_copy(src_hbm.at[wid], src_v)
        pltpu.sync_copy(dst_hbm.at[wid], dst_v)
        plsc.subcore_barrier()

        def chunk(j, carry):
            pltpu.sync_copy(tbl_hbm.at[src_v.at[j]], rows_v)
            pltpu.sync_copy(rows_v, agg_sh.at[dst_v.at[j]], add=True)
            return carry

        lax.fori_loop(0, K_CH, chunk, 0)
        plsc.subcore_barrier()
        pltpu.sync_copy(agg_sh.at[pl.ds(r0, ROWS_PER_TILE)],
                        out_hbm.at[c, pl.ds(r0, ROWS_PER_TILE)])

    return pl.kernel(
        body,
        out_type=jax.ShapeDtypeStruct((NC, N_PAD, F), jnp.float32),
        mesh=mesh,
        scratch_types=[
            pltpu.VMEM((K_CH, CHUNK), jnp.int32),
            pltpu.VMEM((K_CH, CHUNK), jnp.int32),
            pltpu.VMEM((CHUNK, F), jnp.float32),
            pltpu.VMEM_SHARED((N_PAD, F), jnp.float32),
        ],
        compiler_params=pltpu.CompilerParams(use_tc_tiling_on_sc=False),
    )


_agg128 = _make_sc_agg(128)
_agg16 = _make_sc_agg(16)


def _prep_body(x_ref, w_ref, dp_ref, hs_ref, dv_ref):
    deg = dp_ref[0, :, 0:1] + dp_ref[1, :, 0:1] + 1.0
    dinv = lax.rsqrt(deg)
    h = jnp.dot(x_ref[...], w_ref[...], preferred_element_type=jnp.float32)
    hs_ref[...] = h * dinv
    dv_ref[...] = jnp.broadcast_to(dinv, (BLK, 128))


def _prep(x_pad, W_in, degp):
    return pl.pallas_call(
        _prep_body,
        grid=(NBLK,),
        in_specs=[
            pl.BlockSpec((BLK, D), lambda i: (i, 0)),
            pl.BlockSpec((D, H), lambda i: (0, 0)),
            pl.BlockSpec((NC, BLK, 16), lambda i: (0, i, 0)),
        ],
        out_specs=[
            pl.BlockSpec((BLK, H), lambda i: (i, 0)),
            pl.BlockSpec((BLK, 128), lambda i: (i, 0)),
        ],
        out_shape=[
            jax.ShapeDtypeStruct((N_PAD, H), jnp.float32),
            jax.ShapeDtypeStruct((N_PAD, 128), jnp.float32),
        ],
    )(x_pad, W_in, degp)


def _mid_body(p_ref, hs_ref, dv_ref, b_ref, w_ref, o_ref):
    dinv = dv_ref[:, 0:1]
    a = (p_ref[0] + p_ref[1] + hs_ref[...]) * dinv + b_ref[...]
    a = jnp.maximum(a, 0.0)
    o_ref[...] = jnp.dot(a, w_ref[...],
                         preferred_element_type=jnp.float32) * dinv


def _mid(p, hs_prev, dinvb, b2d, W, F_out):
    return pl.pallas_call(
        _mid_body,
        grid=(NBLK,),
        in_specs=[
            pl.BlockSpec((NC, BLK, H), lambda i: (0, i, 0)),
            pl.BlockSpec((BLK, H), lambda i: (i, 0)),
            pl.BlockSpec((BLK, 128), lambda i: (i, 0)),
            pl.BlockSpec((1, H), lambda i: (0, 0)),
            pl.BlockSpec((H, F_out), lambda i: (0, 0)),
        ],
        out_specs=pl.BlockSpec((BLK, F_out), lambda i: (i, 0)),
        out_shape=jax.ShapeDtypeStruct((N_PAD, F_out), jnp.float32),
    )(p, hs_prev, dinvb, b2d, W)


def _pool_body(p_ref, hs_ref, dv_ref, b_ref, bt_ref, o_ref, sums_ref, cnt_ref):
    i = pl.program_id(0)

    @pl.when(i == 0)
    def _():
        sums_ref[...] = jnp.zeros_like(sums_ref)
        cnt_ref[...] = jnp.zeros_like(cnt_ref)

    dinv = dv_ref[:, 0:1]
    h3 = (p_ref[0] + p_ref[1] + hs_ref[...]) * dinv + b_ref[...]
    seg = bt_ref[0, 0, :]
    oh = (seg[:, None] == lax.broadcasted_iota(jnp.int32, (1, G), 1)
          ).astype(jnp.float32)
    sums_ref[...] += lax.dot_general(
        oh, h3, (((0,), (0,)), ((), ())), preferred_element_type=jnp.float32)
    cnt_ref[...] += lax.dot_general(
        oh, jnp.ones_like(h3), (((0,), (0,)), ((), ())),
        preferred_element_type=jnp.float32)

    @pl.when(i == pl.num_programs(0) - 1)
    def _():
        pooled = sums_ref[...] / jnp.maximum(cnt_ref[...], 1.0)
        m = jnp.max(pooled, axis=1, keepdims=True)
        ex = jnp.exp(pooled - m)
        o_ref[...] = (pooled - m) - jnp.log(jnp.sum(ex, axis=1, keepdims=True))


def _pool(p3, hs3, dinvb, b2d, batch3d):
    return pl.pallas_call(
        _pool_body,
        grid=(NBLK,),
        in_specs=[
            pl.BlockSpec((NC, BLK, C), lambda i: (0, i, 0)),
            pl.BlockSpec((BLK, C), lambda i: (i, 0)),
            pl.BlockSpec((BLK, 128), lambda i: (i, 0)),
            pl.BlockSpec((1, C), lambda i: (0, 0)),
            pl.BlockSpec((1, 1, BLK), lambda i: (i, 0, 0)),
        ],
        out_specs=pl.BlockSpec((G, C), lambda i: (0, 0)),
        out_shape=jax.ShapeDtypeStruct((G, C), jnp.float32),
        scratch_shapes=[
            pltpu.VMEM((G, C), jnp.float32),
            pltpu.VMEM((G, C), jnp.float32),
        ],
    )(p3, hs3, dinvb, b2d, batch3d)


def kernel(x, edge_index, batch, W_in, b_in, W_h, b_h, W_out, b_out):
    x_pad = jnp.zeros((N_PAD, D), jnp.float32).at[:N].set(x)
    pad_e = jnp.full((E_PAD - E,), DUMMY, jnp.int32)
    src = jnp.concatenate([edge_index[0], pad_e]).reshape(NW, K_CH, CHUNK)
    dst = jnp.concatenate([edge_index[1], pad_e]).reshape(NW, K_CH, CHUNK)
    batch3d = jnp.concatenate(
        [batch, jnp.full((N_PAD - N,), G, jnp.int32)]).reshape(NBLK, 1, BLK)
    zer128 = jnp.zeros((ROWS_PER_TILE, 128), jnp.float32)
    zer16 = jnp.zeros((ROWS_PER_TILE, 16), jnp.float32)
    ones_tbl = jnp.ones((N_PAD, 16), jnp.float32)

    degp = _agg16(ones_tbl, src, dst, zer16)
    hs1, dinvb = _prep(x_pad, W_in, degp)
    p1 = _agg128(hs1, src, dst, zer128)
    hs2 = _mid(p1, hs1, dinvb, b_in.reshape(1, H), W_h, H)
    p2 = _agg128(hs2, src, dst, zer128)
    hs3 = _mid(p2, hs2, dinvb, b_h.reshape(1, H), W_out, C)
    p3 = _agg16(hs3, src, dst, zer16)
    return _pool(p3, hs3, dinvb, b_out.reshape(1, C), batch3d)
